# baseline (speedup 1.0000x reference)
import functools

import jax
import jax.numpy as jnp
from jax.experimental import pallas as pl
from jax.experimental.pallas import tpu as pltpu
from jax.experimental.pallas import tpu_sc as plsc

_N = 36864
_K = 1024
_D = 64
_BLOCK_IDX = 3072
_BLOCK_PROBS = 4096
_NUM_WORKERS = 32
_ROWS_PER_WORKER = _N // _NUM_WORKERS


def _idx_body(x_ref, w_ref, idx_ref):
    x = x_ref[...]
    w = w_ref[...]
    xsq = jnp.sum(x * x, axis=1, keepdims=True)
    wsq = jnp.sum(w * w, axis=1)
    xw = jax.lax.dot_general(
        x, w, (((1,), (1,)), ((), ())), preferred_element_type=jnp.float32
    )
    dist = xsq + wsq - 2.0 * xw
    iota = jax.lax.broadcasted_iota(jnp.int32, dist.shape, 1)
    minval = jnp.min(dist, axis=1, keepdims=True)
    idx_ref[...] = jnp.min(
        jnp.where(dist == minval, iota, _K), axis=1, keepdims=True
    )


def _probs_body(idx_ref, probs_ref):
    idx = idx_ref[...]
    iota = jax.lax.broadcasted_iota(jnp.int32, (idx.shape[0], _K), 1)
    probs_ref[...] = (iota == idx).astype(jnp.float32)


_SC_CHUNKS = 2
_SC_CHUNK_ROWS = _ROWS_PER_WORKER // _SC_CHUNKS
_DPAD = 128


def _sc_gather_body(table_hbm, idx_hbm, out_hbm, idx_v, rows_v, sem):
    wid = jax.lax.axis_index("s") * 2 + jax.lax.axis_index("c")
    for c in range(_SC_CHUNKS):
        base = wid * _ROWS_PER_WORKER + c * _SC_CHUNK_ROWS
        pltpu.sync_copy(idx_hbm.at[pl.ds(base, _SC_CHUNK_ROWS)], idx_v)
        pltpu.async_copy(table_hbm.at[idx_v], rows_v, sem).wait()
        pltpu.sync_copy(rows_v, out_hbm.at[pl.ds(base, _SC_CHUNK_ROWS)])


def _sc_gather(table_padded, idx_flat):
    mesh = plsc.VectorSubcoreMesh(core_axis_name="c", subcore_axis_name="s")
    run = functools.partial(
        pl.kernel,
        mesh=mesh,
        out_type=jax.ShapeDtypeStruct((_N, _DPAD), jnp.float32),
        scratch_types=[
            pltpu.VMEM((_SC_CHUNK_ROWS,), jnp.int32),
            pltpu.VMEM((_SC_CHUNK_ROWS, _DPAD), jnp.float32),
            pltpu.SemaphoreType.DMA,
        ],
    )(_sc_gather_body)
    return run(table_padded, idx_flat)


def kernel(x, W):
    n, d = x.shape
    k = W.shape[0]
    idx = pl.pallas_call(
        _idx_body,
        grid=(n // _BLOCK_IDX,),
        in_specs=[
            pl.BlockSpec((_BLOCK_IDX, d), lambda i: (i, 0)),
            pl.BlockSpec((k, d), lambda i: (0, 0)),
        ],
        out_specs=pl.BlockSpec((_BLOCK_IDX, 1), lambda i: (i, 0)),
        out_shape=jax.ShapeDtypeStruct((n, 1), jnp.int32),
    )(x, W)
    w_padded = jnp.pad(W, ((0, 0), (0, _DPAD - d)))
    zq = _sc_gather(w_padded, idx.reshape(-1))[:, :d]
    probs = pl.pallas_call(
        _probs_body,
        grid=(n // _BLOCK_PROBS,),
        in_specs=[pl.BlockSpec((_BLOCK_PROBS, 1), lambda i: (i, 0))],
        out_specs=pl.BlockSpec((_BLOCK_PROBS, k), lambda i: (i, 0)),
        out_shape=jax.ShapeDtypeStruct((n, k), jnp.float32),
    )(idx)
    return (zq, probs)


# speedup vs baseline: 1.3815x; 1.3815x over previous
import functools

import jax
import jax.numpy as jnp
from jax.experimental import pallas as pl
from jax.experimental.pallas import tpu as pltpu

_N = 36864
_K = 1024
_D = 64
_BLOCK = 4096


def _body(x_ref, w_ref, zq_ref, probs_ref):
    x = x_ref[...]
    w = w_ref[...]
    xsq = jnp.sum(x * x, axis=1, keepdims=True)
    wsq = jnp.sum(w * w, axis=1)
    xw = jax.lax.dot_general(
        x, w, (((1,), (1,)), ((), ())), preferred_element_type=jnp.float32
    )
    dist = xsq + wsq - 2.0 * xw
    iota = jax.lax.broadcasted_iota(jnp.int32, dist.shape, 1)
    minval = jnp.min(dist, axis=1, keepdims=True)
    idx = jnp.min(jnp.where(dist == minval, iota, _K), axis=1, keepdims=True)
    onehot = (iota == idx).astype(jnp.float32)
    probs_ref[...] = onehot
    dn = (((1,), (0,)), ((), ()))
    zq_ref[...] = jax.lax.dot_general(
        onehot, w, dn, preferred_element_type=jnp.float32
    )


@functools.partial(jax.jit, static_argnames=())
def kernel(x, W):
    n, d = x.shape
    k = W.shape[0]
    grid = (n // _BLOCK,)
    zq, probs = pl.pallas_call(
        _body,
        grid=grid,
        in_specs=[
            pl.BlockSpec((_BLOCK, d), lambda i: (i, 0)),
            pl.BlockSpec((k, d), lambda i: (0, 0)),
        ],
        out_specs=[
            pl.BlockSpec((_BLOCK, d), lambda i: (i, 0)),
            pl.BlockSpec((_BLOCK, k), lambda i: (i, 0)),
        ],
        out_shape=[
            jax.ShapeDtypeStruct((n, d), jnp.float32),
            jax.ShapeDtypeStruct((n, k), jnp.float32),
        ],
        compiler_params=pltpu.CompilerParams(
            vmem_limit_bytes=120 * 1024 * 1024,
        ),
    )(x, W)
    return (zq, probs)


# speedup vs baseline: 1.3874x; 1.0042x over previous
import functools

import jax
import jax.numpy as jnp
from jax.experimental import pallas as pl
from jax.experimental.pallas import tpu as pltpu

_N = 36864
_K = 1024
_D = 64
_BLOCK = 3072


def _body(x_ref, w_ref, zq_ref, probs_ref):
    x = x_ref[...]
    w = w_ref[...]
    xsq = jnp.sum(x * x, axis=1, keepdims=True)
    wsq = jnp.sum(w * w, axis=1)
    xw = jax.lax.dot_general(
        x, w, (((1,), (1,)), ((), ())), preferred_element_type=jnp.float32
    )
    dist = xsq + wsq - 2.0 * xw
    iota = jax.lax.broadcasted_iota(jnp.int32, dist.shape, 1)
    minval = jnp.min(dist, axis=1, keepdims=True)
    idx = jnp.min(jnp.where(dist == minval, iota, _K), axis=1, keepdims=True)
    onehot = (iota == idx).astype(jnp.float32)
    probs_ref[...] = onehot
    dn = (((1,), (0,)), ((), ()))
    zq_ref[...] = jax.lax.dot_general(
        onehot, w, dn, preferred_element_type=jnp.float32
    )


@functools.partial(jax.jit, static_argnames=())
def kernel(x, W):
    n, d = x.shape
    k = W.shape[0]
    grid = (n // _BLOCK,)
    zq, probs = pl.pallas_call(
        _body,
        grid=grid,
        in_specs=[
            pl.BlockSpec((_BLOCK, d), lambda i: (i, 0)),
            pl.BlockSpec((k, d), lambda i: (0, 0)),
        ],
        out_specs=[
            pl.BlockSpec((_BLOCK, d), lambda i: (i, 0)),
            pl.BlockSpec((_BLOCK, k), lambda i: (i, 0)),
        ],
        out_shape=[
            jax.ShapeDtypeStruct((n, d), jnp.float32),
            jax.ShapeDtypeStruct((n, k), jnp.float32),
        ],
        compiler_params=pltpu.CompilerParams(
            vmem_limit_bytes=120 * 1024 * 1024,
        ),
    )(x, W)
    return (zq, probs)


# speedup vs baseline: 1.4599x; 1.0523x over previous
import functools

import jax
import jax.numpy as jnp
from jax.experimental import pallas as pl
from jax.experimental.pallas import tpu as pltpu

_N = 36864
_K = 1024
_D = 64
_BLOCK = 3072


def _body(x_ref, w_ref, zq_ref, probs_ref):
    x = x_ref[...]
    w = w_ref[...]
    xsq = jnp.sum(x * x, axis=1, keepdims=True)
    wsq = jnp.sum(w * w, axis=1)
    xw = jax.lax.dot_general(
        x, w, (((1,), (1,)), ((), ())), preferred_element_type=jnp.float32
    )
    dist = xsq + wsq - 2.0 * xw
    ncols = _K // 128
    cols = [dist[:, j * 128:(j + 1) * 128] for j in range(ncols)]
    vt = list(cols)
    jt = [jnp.full(cols[0].shape, j, jnp.int32) for j in range(ncols)]
    while len(vt) > 1:
        nv, nj = [], []
        for p in range(0, len(vt), 2):
            take_b = vt[p + 1] < vt[p]
            nv.append(jnp.minimum(vt[p], vt[p + 1]))
            nj.append(jnp.where(take_b, jt[p + 1], jt[p]))
        vt, jt = nv, nj
    lane = jax.lax.broadcasted_iota(jnp.int32, cols[0].shape, 1)
    gidx = jt[0] * 128 + lane
    minval = jnp.min(vt[0], axis=1, keepdims=True)
    idx = jnp.min(jnp.where(vt[0] == minval, gidx, _K), axis=1, keepdims=True)
    iota = jax.lax.broadcasted_iota(jnp.int32, dist.shape, 1)
    onehot = (iota == idx).astype(jnp.float32)
    probs_ref[...] = onehot
    dn = (((1,), (0,)), ((), ()))
    zq_ref[...] = jax.lax.dot_general(
        onehot, w, dn, preferred_element_type=jnp.float32
    )


@functools.partial(jax.jit, static_argnames=())
def kernel(x, W):
    n, d = x.shape
    k = W.shape[0]
    grid = (n // _BLOCK,)
    zq, probs = pl.pallas_call(
        _body,
        grid=grid,
        in_specs=[
            pl.BlockSpec((_BLOCK, d), lambda i: (i, 0)),
            pl.BlockSpec((k, d), lambda i: (0, 0)),
        ],
        out_specs=[
            pl.BlockSpec((_BLOCK, d), lambda i: (i, 0)),
            pl.BlockSpec((_BLOCK, k), lambda i: (i, 0)),
        ],
        out_shape=[
            jax.ShapeDtypeStruct((n, d), jnp.float32),
            jax.ShapeDtypeStruct((n, k), jnp.float32),
        ],
        compiler_params=pltpu.CompilerParams(
            vmem_limit_bytes=120 * 1024 * 1024,
        ),
    )(x, W)
    return (zq, probs)
